# Initial kernel scaffold; baseline (speedup 1.0000x reference)
import jax
import jax.numpy as jnp
from jax.experimental import pallas as pl


def kernel(out, z_rv, x_rv_batch):
    raise NotImplementedError("write your pallas kernel here")


# speedup vs baseline: 11.0196x; 11.0196x over previous
import functools

import jax
import jax.numpy as jnp
from jax import lax
from jax.experimental import pallas as pl
from jax.experimental.pallas import tpu as pltpu
from jax.experimental.pallas import tpu_sc as plsc

N = 320000
D = 128
G = 128
VIRT = 100
NC = 2
NS = 16
NW = NC * NS
RW = N // NW
C = 80
NCH = RW // C


@functools.partial(
    pl.kernel,
    mesh=plsc.VectorSubcoreMesh(core_axis_name="c", subcore_axis_name="s"),
    out_type=jax.ShapeDtypeStruct((NC, 2 * G, D), jnp.float32),
    scratch_types=[
        pltpu.VMEM((RW,), jnp.int32),
        pltpu.VMEM((RW,), jnp.int32),
        pltpu.VMEM((C, D), jnp.float32),
        pltpu.VMEM((C, D), jnp.float32),
        pltpu.VMEM((C,), jnp.int32),
        pltpu.VMEM((C,), jnp.int32),
        pltpu.VMEM((16, D), jnp.float32),
        pltpu.VMEM_SHARED((2 * G, D), jnp.float32),
        pltpu.SemaphoreType.DMA,
        pltpu.SemaphoreType.DMA,
    ],
)
def _pool_kernel(x_hbm, z_hbm, b_hbm, out_hbm,
                 z_v, b_v, row0, row1, dst0, dst1, zbuf, acc, sem0, sem1):
    cid = lax.axis_index("c")
    sid = lax.axis_index("s")
    wid = cid * NS + sid
    base = wid * RW

    zeros16 = jnp.zeros((16,), jnp.float32)
    for r in range(16):
        for k in range(D // 16):
            zbuf[r, pl.ds(k * 16, 16)] = zeros16
    pltpu.sync_copy(zbuf, acc.at[pl.ds(sid * 16, 16)])
    plsc.subcore_barrier()

    pltpu.sync_copy(z_hbm.at[pl.ds(base, RW)], z_v)
    pltpu.sync_copy(b_hbm.at[pl.ds(base, RW)], b_v)

    rows = (row0, row1)
    dsts = (dst0, dst1)
    sems = (sem0, sem1)

    def start(j, b):
        pltpu.make_async_copy(
            x_hbm.at[pl.ds(base + j * C, C)], rows[b], sems[b]).start()

    def drain(b):
        pltpu.make_async_copy(
            x_hbm.at[pl.ds(0, C)], rows[b], sems[b]).wait()

    def process(j, b):
        for k in range(C // 16):
            off = j * C + k * 16
            zk = z_v[pl.ds(off, 16)]
            bk = b_v[pl.ds(off, 16)]
            dk = bk * 2 + jnp.where(zk == VIRT, 1, 0).astype(jnp.int32)
            dsts[b][pl.ds(k * 16, 16)] = dk
        drain(b)
        pltpu.sync_copy(rows[b], acc.at[dsts[b]], add=True)

    start(0, 0)

    def body(t, carry):
        start(2 * t + 1, 1)
        process(2 * t, 0)
        start(2 * t + 2, 0)
        process(2 * t + 1, 1)
        return carry

    lax.fori_loop(0, (NCH - 1) // 2, body, 0)
    process(NCH - 1, 0)

    plsc.subcore_barrier()

    @pl.when(sid == 0)
    def _():
        pltpu.sync_copy(acc, out_hbm.at[cid])


def kernel(out, z_rv, x_rv_batch):
    part = _pool_kernel(out,
                        z_rv.astype(jnp.int32),
                        x_rv_batch.astype(jnp.int32))
    return (part[0] + part[1]).reshape(G, 2 * D)
